# Initial kernel scaffold; baseline (speedup 1.0000x reference)
import jax
import jax.numpy as jnp
from jax.experimental import pallas as pl


def kernel(x, position_ids, cos_cached, sin_cached):
    raise NotImplementedError("write your pallas kernel here")


# speedup vs baseline: 3.4576x; 3.4576x over previous
import functools

import jax
import jax.numpy as jnp
from jax import lax
from jax.experimental import pallas as pl
from jax.experimental.pallas import tpu as pltpu
from jax.experimental.pallas import tpu_sc as plsc

_INFO = plsc.get_sparse_core_info()
_NC = _INFO.num_cores
_NS = _INFO.num_subcores
_NW = _NC * _NS


@functools.lru_cache(maxsize=None)
def _build_gather(n_idx: int, dim: int):
    assert n_idx % (8 * _NW) == 0
    b_per_w = n_idx // _NW
    mesh = plsc.VectorSubcoreMesh(core_axis_name="c", subcore_axis_name="s")

    @functools.partial(
        pl.kernel,
        mesh=mesh,
        out_type=(
            jax.ShapeDtypeStruct((n_idx, dim), jnp.float32),
            jax.ShapeDtypeStruct((n_idx, dim), jnp.float32),
        ),
        scratch_types=[
            pltpu.VMEM((b_per_w,), jnp.int32),
            pltpu.VMEM((b_per_w, dim), jnp.float32),
            pltpu.SemaphoreType.DMA,
        ],
    )
    def gather(cos_hbm, sin_hbm, idx_hbm, cos_out, sin_out, idx_v, rows_v, sem):
        wid = lax.axis_index("s") * _NC + lax.axis_index("c")
        base = wid * b_per_w
        pltpu.sync_copy(idx_hbm.at[pl.ds(base, b_per_w)], idx_v)
        pltpu.async_copy(cos_hbm.at[idx_v], rows_v, sem).wait()
        pltpu.sync_copy(rows_v, cos_out.at[pl.ds(base, b_per_w)])
        pltpu.async_copy(sin_hbm.at[idx_v], rows_v, sem).wait()
        pltpu.sync_copy(rows_v, sin_out.at[pl.ds(base, b_per_w)])

    return gather


def kernel(x, position_ids, cos_cached, sin_cached):
    b, s = position_ids.shape
    dim = cos_cached.shape[-1]
    idx = position_ids.reshape(-1).astype(jnp.int32)
    cos, sin = _build_gather(b * s, dim)(cos_cached, sin_cached, idx)
    return (
        cos.reshape(b, s, dim).astype(x.dtype),
        sin.reshape(b, s, dim).astype(x.dtype),
    )


# speedup vs baseline: 3.5960x; 1.0400x over previous
import functools

import jax
import jax.numpy as jnp
from jax import lax
from jax.experimental import pallas as pl
from jax.experimental.pallas import tpu as pltpu
from jax.experimental.pallas import tpu_sc as plsc

_INFO = plsc.get_sparse_core_info()
_NC = _INFO.num_cores
_NS = _INFO.num_subcores
_NW = _NC * _NS
_CHUNKS = 4
_NBUF = 4
_LAG = 3


@functools.lru_cache(maxsize=None)
def _build_gather(n_idx: int, dim: int):
    b_per_w = n_idx // _NW
    rows = b_per_w // _CHUNKS
    assert rows * _CHUNKS * _NW == n_idx and rows % 8 == 0
    n_tasks = 2 * _CHUNKS
    mesh = plsc.VectorSubcoreMesh(core_axis_name="c", subcore_axis_name="s")

    @functools.partial(
        pl.kernel,
        mesh=mesh,
        out_type=(
            jax.ShapeDtypeStruct((n_idx, dim), jnp.float32),
            jax.ShapeDtypeStruct((n_idx, dim), jnp.float32),
        ),
        scratch_types=[
            pltpu.VMEM((_CHUNKS, rows), jnp.int32),
            pltpu.VMEM((_NBUF, rows, dim), jnp.float32),
            pltpu.SemaphoreType.DMA((_NBUF,)),
            pltpu.SemaphoreType.DMA((_NBUF,)),
        ],
    )
    def gather(cos_hbm, sin_hbm, idx_hbm, cos_out, sin_out,
               idx_v, bufs, gsems, wsems):
        wid = lax.axis_index("s") * _NC + lax.axis_index("c")
        base = wid * b_per_w
        pltpu.sync_copy(idx_hbm.at[wid], idx_v)

        def task(t):
            tbl = cos_hbm if t < _CHUNKS else sin_hbm
            out = cos_out if t < _CHUNKS else sin_out
            return tbl, out, t % _CHUNKS

        gh = {}
        wh = {}
        waited = set()

        def start_gather(t):
            tbl, _, c = task(t)
            b = t % _NBUF
            gh[t] = pltpu.async_copy(tbl.at[idx_v.at[c]], bufs.at[b],
                                     gsems.at[b])

        for t in range(_LAG):
            start_gather(t)
        for t in range(n_tasks):
            nxt = t + _LAG
            if nxt < n_tasks:
                prev = nxt - _NBUF
                if prev >= 0:
                    wh[prev].wait()
                    waited.add(prev)
                start_gather(nxt)
            _, out, c = task(t)
            b = t % _NBUF
            gh[t].wait()
            wh[t] = pltpu.async_copy(bufs.at[b],
                                     out.at[pl.ds(base + c * rows, rows)],
                                     wsems.at[b])
        for t in range(n_tasks):
            if t not in waited:
                wh[t].wait()

    return gather


def kernel(x, position_ids, cos_cached, sin_cached):
    b, s = position_ids.shape
    dim = cos_cached.shape[-1]
    n_idx = b * s
    rows = n_idx // (_NW * _CHUNKS)
    idx = position_ids.reshape(_NW, _CHUNKS, rows).astype(jnp.int32)
    cos, sin = _build_gather(n_idx, dim)(cos_cached, sin_cached, idx)
    return (
        cos.reshape(b, s, dim).astype(x.dtype),
        sin.reshape(b, s, dim).astype(x.dtype),
    )


# speedup vs baseline: 3.6221x; 1.0072x over previous
import functools

import jax
import jax.numpy as jnp
from jax import lax
from jax.experimental import pallas as pl
from jax.experimental.pallas import tpu as pltpu
from jax.experimental.pallas import tpu_sc as plsc

_INFO = plsc.get_sparse_core_info()
_NC = _INFO.num_cores
_NS = _INFO.num_subcores
_NW = _NC * _NS
_CHUNKS = 4
_NBUF = 4
_LAG = 3


@functools.lru_cache(maxsize=None)
def _build_gather(nb: int, s: int, dim: int):
    w_per_b = _NW // nb
    b_per_w = s // w_per_b
    rows = b_per_w // _CHUNKS
    assert rows * _CHUNKS * w_per_b == s and rows % 8 == 0
    n_tasks = 2 * _CHUNKS
    mesh = plsc.VectorSubcoreMesh(core_axis_name="c", subcore_axis_name="s")

    @functools.partial(
        pl.kernel,
        mesh=mesh,
        out_type=(
            jax.ShapeDtypeStruct((nb, s, dim), jnp.float32),
            jax.ShapeDtypeStruct((nb, s, dim), jnp.float32),
        ),
        scratch_types=[
            pltpu.VMEM((b_per_w,), jnp.int32),
            pltpu.VMEM((_NBUF, rows, dim), jnp.float32),
            pltpu.SemaphoreType.DMA((_NBUF,)),
            pltpu.SemaphoreType.DMA((_NBUF,)),
        ],
    )
    def gather(cos_hbm, sin_hbm, idx_hbm, cos_out, sin_out,
               idx_v, bufs, gsems, wsems):
        wid = lax.axis_index("s") * _NC + lax.axis_index("c")
        bi = wid // w_per_b
        off = (wid % w_per_b) * b_per_w
        pltpu.sync_copy(idx_hbm.at[bi, pl.ds(off, b_per_w)], idx_v)

        def task(t):
            tbl = cos_hbm if t < _CHUNKS else sin_hbm
            out = cos_out if t < _CHUNKS else sin_out
            return tbl, out, t % _CHUNKS

        gh = {}
        wh = {}
        waited = set()

        def start_gather(t):
            tbl, _, c = task(t)
            b = t % _NBUF
            gh[t] = pltpu.async_copy(
                tbl.at[idx_v.at[pl.ds(c * rows, rows)]], bufs.at[b],
                gsems.at[b])

        for t in range(_LAG):
            start_gather(t)
        for t in range(n_tasks):
            nxt = t + _LAG
            if nxt < n_tasks:
                prev = nxt - _NBUF
                if prev >= 0:
                    wh[prev].wait()
                    waited.add(prev)
                start_gather(nxt)
            _, out, c = task(t)
            b = t % _NBUF
            gh[t].wait()
            wh[t] = pltpu.async_copy(
                bufs.at[b], out.at[bi, pl.ds(off + c * rows, rows)],
                wsems.at[b])
        for t in range(n_tasks):
            if t not in waited:
                wh[t].wait()

    return gather


def kernel(x, position_ids, cos_cached, sin_cached):
    nb, s = position_ids.shape
    dim = cos_cached.shape[-1]
    cos, sin = _build_gather(nb, s, dim)(
        cos_cached, sin_cached, position_ids.astype(jnp.int32))
    return cos.astype(x.dtype), sin.astype(x.dtype)
